# Initial kernel scaffold; baseline (speedup 1.0000x reference)
import jax
import jax.numpy as jnp
from jax.experimental import pallas as pl


def kernel(nodes, edge_features, distance, edges, node_mask, W_msg, b_msg, ln1_g, ln1_b, Wq, bq, Wk, bk, Wv, bv, Wo, bo, ln2_g, ln2_b, sigma, beta):
    raise NotImplementedError("write your pallas kernel here")


# speedup vs baseline: 6.3211x; 6.3211x over previous
import functools
import jax
import jax.numpy as jnp
from jax.experimental import pallas as pl
from jax.experimental.pallas import tpu as pltpu

_F32 = jnp.float32
_BF16 = jnp.bfloat16
_H = 8


def _erf(x):
    p = 0.3275911
    a1, a2, a3, a4, a5 = (0.254829592, -0.284496736, 1.421413741,
                          -1.453152027, 1.061405429)
    ax = jnp.abs(x)
    t = 1.0 / (1.0 + p * ax)
    poly = ((((a5 * t + a4) * t + a3) * t + a2) * t + a1) * t
    y = 1.0 - poly * jnp.exp(-ax * ax)
    return jnp.sign(x) * y


def _gelu(x):
    return 0.5 * x * (1.0 + _erf(x * 0.7071067811865475))


def _ln(x, g, b, eps=1e-3):
    mu = jnp.mean(x, axis=-1, keepdims=True)
    var = jnp.mean((x - mu) ** 2, axis=-1, keepdims=True)
    return (x - mu) / jnp.sqrt(var + eps) * g + b


def _prep_body(D, nodes_ref, w_ref, t_ref):
    n = nodes_ref[0]
    N = n.shape[0]
    t_ref[0, :N] = jnp.dot(n, w_ref[0:D], preferred_element_type=_F32
                           ).astype(_BF16)
    t_ref[0, N:] = jnp.dot(n, w_ref[D:2 * D], preferred_element_type=_F32
                           ).astype(_BF16)


def _edge_body(t_ref, src_ref, dst_ref, ef_ref, dist_ref, we_ref, bm_ref,
               g1_ref, b1_ref, sg_ref, bt_ref, wm_ref, ew_ref, agg_ref):
    twoN = t_ref.shape[1]
    N = twoN // 2
    Eb = src_ref.shape[-1]
    src = src_ref[0, 0, 0]
    dst = dst_ref[0, 0, 0]
    ids = jax.lax.broadcasted_iota(jnp.int32, (Eb, twoN), 1)
    sel = (ids == src[:, None]) | (ids == (dst[:, None] + N))
    oh = sel.astype(_BF16)
    G = jnp.dot(oh, t_ref[0], preferred_element_type=_F32)
    R = jnp.dot(ef_ref[0], we_ref[...], preferred_element_type=_F32)
    pre = G + R + bm_ref[0]
    m = _ln(_gelu(pre), g1_ref[0], b1_ref[0])
    d = dist_ref[0, 0, 0]
    sig = sg_ref[0, 0]
    bet = bt_ref[0, 0]
    z = d * d / (2.0 * sig * sig)
    zb = jnp.exp(bet * jnp.log(jnp.maximum(z, 1e-38)))
    ew = jnp.exp(-zb)
    wm = m * ew[:, None]
    wm_ref[0] = wm
    ew_ref[0, 0, 0] = ew
    rowids = jax.lax.broadcasted_iota(jnp.int32, (N, Eb), 0)
    ohT = (rowids == dst[None, :]).astype(_BF16)
    part = jnp.dot(ohT, wm.astype(_BF16), preferred_element_type=_F32)
    eb = pl.program_id(1)

    @pl.when(eb == 0)
    def _():
        agg_ref[0] = part

    @pl.when(eb > 0)
    def _():
        agg_ref[0] = agg_ref[0] + part


def _attn_body(F, nodes_ref, agg_ref, wq_ref, bq_ref, wk_ref, bk_ref,
               wv_ref, bv_ref, wo_ref, bo_ref, g2_ref, b2_ref, out_ref):
    x = jnp.concatenate([nodes_ref[0], agg_ref[0]], axis=1)
    q = jnp.dot(x, wq_ref[...], preferred_element_type=_F32) + bq_ref[0]
    k = jnp.dot(x, wk_ref[...], preferred_element_type=_F32) + bk_ref[0]
    v = jnp.dot(x, wv_ref[...], preferred_element_type=_F32) + bv_ref[0]
    pd = F // _H
    scale = 1.0 / (pd ** 0.5)
    outs = []
    for h in range(_H):
        sl = slice(h * pd, (h + 1) * pd)
        qh, kh, vh = q[:, sl], k[:, sl], v[:, sl]
        s = jax.lax.dot_general(qh, kh, (((1,), (1,)), ((), ())),
                                preferred_element_type=_F32) * scale
        s = s - jnp.max(s, axis=-1, keepdims=True)
        e = jnp.exp(s)
        w = e / jnp.sum(e, axis=-1, keepdims=True)
        outs.append(jnp.dot(w, vh, preferred_element_type=_F32))
    att = jnp.concatenate(outs, axis=1)
    o = jnp.dot(att, wo_ref[...], preferred_element_type=_F32) + bo_ref[0]
    out_ref[0] = _ln(_gelu(o), g2_ref[0], b2_ref[0])


def kernel(nodes, edge_features, distance, edges, node_mask, W_msg, b_msg,
           ln1_g, ln1_b, Wq, bq, Wk, bk, Wv, bv, Wo, bo, ln2_g, ln2_b,
           sigma, beta):
    B, N, D = nodes.shape
    E = edges.shape[1]
    DE = edge_features.shape[2]
    F = W_msg.shape[1]
    Eb = min(512, E)
    nblk = E // Eb

    T = pl.pallas_call(
        functools.partial(_prep_body, D),
        grid=(B,),
        in_specs=[
            pl.BlockSpec((1, N, D), lambda b: (b, 0, 0)),
            pl.BlockSpec((2 * D + DE, F), lambda b: (0, 0)),
        ],
        out_specs=pl.BlockSpec((1, 2 * N, F), lambda b: (b, 0, 0)),
        out_shape=jax.ShapeDtypeStruct((B, 2 * N, F), _BF16),
    )(nodes, W_msg)

    src4 = edges[:, :, 0].reshape(B, nblk, 1, Eb)
    dst4 = edges[:, :, 1].reshape(B, nblk, 1, Eb)
    dist4 = distance.reshape(B, nblk, 1, Eb)
    ef8 = jnp.pad(edge_features, ((0, 0), (0, 0), (0, 8 - DE)))
    We8 = jnp.zeros((8, F), _F32).at[:DE].set(W_msg[2 * D:])
    bm = b_msg.reshape(1, F)
    g1, b1 = ln1_g.reshape(1, F), ln1_b.reshape(1, F)
    g2, b2 = ln2_g.reshape(1, F), ln2_b.reshape(1, F)
    sg, bt = sigma.reshape(1, 1), beta.reshape(1, 1)

    wm, ew4, agg = pl.pallas_call(
        _edge_body,
        grid=(B, nblk),
        in_specs=[
            pl.BlockSpec((1, 2 * N, F), lambda b, e: (b, 0, 0)),
            pl.BlockSpec((1, 1, 1, Eb), lambda b, e: (b, e, 0, 0)),
            pl.BlockSpec((1, 1, 1, Eb), lambda b, e: (b, e, 0, 0)),
            pl.BlockSpec((1, Eb, 8), lambda b, e: (b, e, 0)),
            pl.BlockSpec((1, 1, 1, Eb), lambda b, e: (b, e, 0, 0)),
            pl.BlockSpec((8, F), lambda b, e: (0, 0)),
            pl.BlockSpec((1, F), lambda b, e: (0, 0)),
            pl.BlockSpec((1, F), lambda b, e: (0, 0)),
            pl.BlockSpec((1, F), lambda b, e: (0, 0)),
            pl.BlockSpec((1, 1), lambda b, e: (0, 0)),
            pl.BlockSpec((1, 1), lambda b, e: (0, 0)),
        ],
        out_specs=[
            pl.BlockSpec((1, Eb, F), lambda b, e: (b, e, 0)),
            pl.BlockSpec((1, 1, 1, Eb), lambda b, e: (b, e, 0, 0)),
            pl.BlockSpec((1, N, F), lambda b, e: (b, 0, 0)),
        ],
        out_shape=[
            jax.ShapeDtypeStruct((B, E, F), _F32),
            jax.ShapeDtypeStruct((B, nblk, 1, Eb), _F32),
            jax.ShapeDtypeStruct((B, N, F), _F32),
        ],
    )(T, src4, dst4, ef8, dist4, We8, bm, g1, b1, sg, bt)

    updated = pl.pallas_call(
        functools.partial(_attn_body, F),
        grid=(B,),
        in_specs=[
            pl.BlockSpec((1, N, D), lambda b: (b, 0, 0)),
            pl.BlockSpec((1, N, F), lambda b: (b, 0, 0)),
            pl.BlockSpec((D + F, F), lambda b: (0, 0)),
            pl.BlockSpec((1, F), lambda b: (0, 0)),
            pl.BlockSpec((D + F, F), lambda b: (0, 0)),
            pl.BlockSpec((1, F), lambda b: (0, 0)),
            pl.BlockSpec((D + F, F), lambda b: (0, 0)),
            pl.BlockSpec((1, F), lambda b: (0, 0)),
            pl.BlockSpec((F, F), lambda b: (0, 0)),
            pl.BlockSpec((1, F), lambda b: (0, 0)),
            pl.BlockSpec((1, F), lambda b: (0, 0)),
            pl.BlockSpec((1, F), lambda b: (0, 0)),
        ],
        out_specs=pl.BlockSpec((1, N, F), lambda b: (b, 0, 0)),
        out_shape=jax.ShapeDtypeStruct((B, N, F), _F32),
    )(nodes, agg, Wq, bq.reshape(1, F), Wk, bk.reshape(1, F),
      Wv, bv.reshape(1, F), Wo, bo.reshape(1, F), g2, b2)

    return (updated, wm, distance, edges, ew4.reshape(B, E))


# speedup vs baseline: 7.1989x; 1.1389x over previous
import functools
import jax
import jax.numpy as jnp
from jax.experimental import pallas as pl
from jax.experimental.pallas import tpu as pltpu

_F32 = jnp.float32
_BF16 = jnp.bfloat16
_H = 8


def _erf(x):
    p = 0.3275911
    a1, a2, a3, a4, a5 = (0.254829592, -0.284496736, 1.421413741,
                          -1.453152027, 1.061405429)
    ax = jnp.abs(x)
    t = 1.0 / (1.0 + p * ax)
    poly = ((((a5 * t + a4) * t + a3) * t + a2) * t + a1) * t
    y = 1.0 - poly * jnp.exp(-ax * ax)
    return jnp.sign(x) * y


def _gelu(x):
    return 0.5 * x * (1.0 + _erf(x * 0.7071067811865475))


def _ln(x, g, b, eps=1e-3):
    mu = jnp.mean(x, axis=-1, keepdims=True)
    var = jnp.mean((x - mu) ** 2, axis=-1, keepdims=True)
    return (x - mu) / jnp.sqrt(var + eps) * g + b


def _prep_body(D, nodes_ref, w_ref, t_ref):
    n = nodes_ref[0]
    N = n.shape[0]
    t_ref[0, :N] = jnp.dot(n, w_ref[0:D], preferred_element_type=_F32
                           ).astype(_BF16)
    t_ref[0, N:] = jnp.dot(n, w_ref[D:2 * D], preferred_element_type=_F32
                           ).astype(_BF16)


def _edge_body(t_ref, edges_ref, ef_ref, dist_ref, we_ref, bm_ref,
               g1_ref, b1_ref, sg_ref, bt_ref, wm_ref, ew_ref, agg_ref):
    twoN = t_ref.shape[1]
    N = twoN // 2
    Eb = edges_ref.shape[2]
    e2 = edges_ref[0, 0]
    src = e2[:, 0]
    dst = e2[:, 1]
    ids = jax.lax.broadcasted_iota(jnp.int32, (Eb, twoN), 1)
    sel = (ids == src[:, None]) | (ids == (dst[:, None] + N))
    oh = sel.astype(_BF16)
    G = jnp.dot(oh, t_ref[0], preferred_element_type=_F32)
    R = jnp.dot(ef_ref[0], we_ref[...], preferred_element_type=_F32)
    pre = G + R + bm_ref[0]
    m = _ln(_gelu(pre), g1_ref[0], b1_ref[0])
    d = dist_ref[0, 0, 0]
    sig = sg_ref[0, 0]
    bet = bt_ref[0, 0]
    z = d * d / (2.0 * sig * sig)
    zb = jnp.exp(bet * jnp.log(jnp.maximum(z, 1e-38)))
    ew = jnp.exp(-zb)
    wm = m * ew[:, None]
    wm_ref[0] = wm
    ew_ref[0, 0, 0] = ew
    rowids = jax.lax.broadcasted_iota(jnp.int32, (N, Eb), 0)
    ohT = (rowids == dst[None, :]).astype(_BF16)
    part = jnp.dot(ohT, wm.astype(_BF16), preferred_element_type=_F32)
    eb = pl.program_id(1)

    @pl.when(eb == 0)
    def _():
        agg_ref[0] = part

    @pl.when(eb > 0)
    def _():
        agg_ref[0] = agg_ref[0] + part


def _attn_body(F, nodes_ref, agg_ref, wq_ref, bq_ref, wk_ref, bk_ref,
               wv_ref, bv_ref, wo_ref, bo_ref, g2_ref, b2_ref, out_ref):
    x = jnp.concatenate([nodes_ref[0], agg_ref[0]], axis=1)
    q = jnp.dot(x, wq_ref[...], preferred_element_type=_F32) + bq_ref[0]
    k = jnp.dot(x, wk_ref[...], preferred_element_type=_F32) + bk_ref[0]
    v = jnp.dot(x, wv_ref[...], preferred_element_type=_F32) + bv_ref[0]
    pd = F // _H
    scale = 1.0 / (pd ** 0.5)
    outs = []
    for h in range(_H):
        sl = slice(h * pd, (h + 1) * pd)
        qh, kh, vh = q[:, sl], k[:, sl], v[:, sl]
        s = jax.lax.dot_general(qh, kh, (((1,), (1,)), ((), ())),
                                preferred_element_type=_F32) * scale
        s = s - jnp.max(s, axis=-1, keepdims=True)
        e = jnp.exp(s)
        w = e / jnp.sum(e, axis=-1, keepdims=True)
        outs.append(jnp.dot(w, vh, preferred_element_type=_F32))
    att = jnp.concatenate(outs, axis=1)
    o = jnp.dot(att, wo_ref[...], preferred_element_type=_F32) + bo_ref[0]
    out_ref[0] = _ln(_gelu(o), g2_ref[0], b2_ref[0])


def kernel(nodes, edge_features, distance, edges, node_mask, W_msg, b_msg,
           ln1_g, ln1_b, Wq, bq, Wk, bk, Wv, bv, Wo, bo, ln2_g, ln2_b,
           sigma, beta):
    B, N, D = nodes.shape
    E = edges.shape[1]
    DE = edge_features.shape[2]
    F = W_msg.shape[1]
    Eb = min(512, E)
    nblk = E // Eb

    T = pl.pallas_call(
        functools.partial(_prep_body, D),
        grid=(B,),
        in_specs=[
            pl.BlockSpec((1, N, D), lambda b: (b, 0, 0)),
            pl.BlockSpec((2 * D + DE, F), lambda b: (0, 0)),
        ],
        out_specs=pl.BlockSpec((1, 2 * N, F), lambda b: (b, 0, 0)),
        out_shape=jax.ShapeDtypeStruct((B, 2 * N, F), _BF16),
    )(nodes, W_msg)

    edges4 = edges.reshape(B, nblk, Eb, 2)
    dist4 = distance.reshape(B, nblk, 1, Eb)
    We = W_msg[2 * D:]
    bm = b_msg.reshape(1, F)
    g1, b1 = ln1_g.reshape(1, F), ln1_b.reshape(1, F)
    g2, b2 = ln2_g.reshape(1, F), ln2_b.reshape(1, F)
    sg, bt = sigma.reshape(1, 1), beta.reshape(1, 1)

    wm, ew4, agg = pl.pallas_call(
        _edge_body,
        grid=(B, nblk),
        in_specs=[
            pl.BlockSpec((1, 2 * N, F), lambda b, e: (b, 0, 0)),
            pl.BlockSpec((1, 1, Eb, 2), lambda b, e: (b, e, 0, 0)),
            pl.BlockSpec((1, Eb, DE), lambda b, e: (b, e, 0)),
            pl.BlockSpec((1, 1, 1, Eb), lambda b, e: (b, e, 0, 0)),
            pl.BlockSpec((DE, F), lambda b, e: (0, 0)),
            pl.BlockSpec((1, F), lambda b, e: (0, 0)),
            pl.BlockSpec((1, F), lambda b, e: (0, 0)),
            pl.BlockSpec((1, F), lambda b, e: (0, 0)),
            pl.BlockSpec((1, 1), lambda b, e: (0, 0)),
            pl.BlockSpec((1, 1), lambda b, e: (0, 0)),
        ],
        out_specs=[
            pl.BlockSpec((1, Eb, F), lambda b, e: (b, e, 0)),
            pl.BlockSpec((1, 1, 1, Eb), lambda b, e: (b, e, 0, 0)),
            pl.BlockSpec((1, N, F), lambda b, e: (b, 0, 0)),
        ],
        out_shape=[
            jax.ShapeDtypeStruct((B, E, F), _F32),
            jax.ShapeDtypeStruct((B, nblk, 1, Eb), _F32),
            jax.ShapeDtypeStruct((B, N, F), _F32),
        ],
    )(T, edges4, edge_features, dist4, We, bm, g1, b1, sg, bt)

    updated = pl.pallas_call(
        functools.partial(_attn_body, F),
        grid=(B,),
        in_specs=[
            pl.BlockSpec((1, N, D), lambda b: (b, 0, 0)),
            pl.BlockSpec((1, N, F), lambda b: (b, 0, 0)),
            pl.BlockSpec((D + F, F), lambda b: (0, 0)),
            pl.BlockSpec((1, F), lambda b: (0, 0)),
            pl.BlockSpec((D + F, F), lambda b: (0, 0)),
            pl.BlockSpec((1, F), lambda b: (0, 0)),
            pl.BlockSpec((D + F, F), lambda b: (0, 0)),
            pl.BlockSpec((1, F), lambda b: (0, 0)),
            pl.BlockSpec((F, F), lambda b: (0, 0)),
            pl.BlockSpec((1, F), lambda b: (0, 0)),
            pl.BlockSpec((1, F), lambda b: (0, 0)),
            pl.BlockSpec((1, F), lambda b: (0, 0)),
        ],
        out_specs=pl.BlockSpec((1, N, F), lambda b: (b, 0, 0)),
        out_shape=jax.ShapeDtypeStruct((B, N, F), _F32),
    )(nodes, agg, Wq, bq.reshape(1, F), Wk, bk.reshape(1, F),
      Wv, bv.reshape(1, F), Wo, bo.reshape(1, F), g2, b2)

    return (updated, wm, distance, edges, ew4.reshape(B, E))
